# baseline (speedup 1.0000x reference)
import functools

import jax
import jax.numpy as jnp
from jax import lax
from jax.experimental import pallas as pl
from jax.experimental.pallas import tpu as pltpu
from jax.experimental.pallas import tpu_sc as plsc

B, N, V, DIN, DOUT = 4096, 50, 1000000, 64, 64
_EW = 128

_NC, _NS = 2, 16
_NW = _NC * _NS
_ROWS = B * N
_PER_W = _ROWS // _NW
_CHUNK = 128
_NCHUNK = _PER_W // _CHUNK


@functools.cache
def _make_sc_gather():
    mesh = plsc.VectorSubcoreMesh(core_axis_name="c", subcore_axis_name="s")

    @functools.partial(
        pl.kernel,
        mesh=mesh,
        compiler_params=pltpu.CompilerParams(
            use_tc_tiling_on_sc=False, needs_layout_passes=False
        ),
        out_type=jax.ShapeDtypeStruct((_ROWS, _EW), jnp.float32),
        scratch_types=[
            pltpu.VMEM((_CHUNK, N), jnp.int32),
            pltpu.VMEM((N, _CHUNK), jnp.int32),
            pltpu.VMEM((2, _CHUNK, DIN), jnp.float32),
            pltpu.SemaphoreType.DMA,
            pltpu.SemaphoreType.DMA,
        ],
    )
    def _sc_gather(x_hbm, table_hbm, out_hbm, xv, idx_v, rows_v, sem0, sem1):
        wid = lax.axis_index("s") * _NC + lax.axis_index("c")
        pltpu.sync_copy(x_hbm.at[wid], xv)

        lanes = lax.iota(jnp.int32, 16)

        def transpose_col(n, carry):
            col = jnp.full((16,), n, jnp.int32)
            for k in range(_CHUNK // 16):
                v = plsc.load_gather(xv, [lanes + 16 * k, col])
                idx_v[n, pl.ds(16 * k, 16)] = v
            return carry

        lax.fori_loop(0, N, transpose_col, 0)

        sems = (sem0, sem1)

        def start(j, slot):
            pltpu.async_copy(
                table_hbm.at[idx_v.at[j]], rows_v.at[slot], sems[slot]
            )

        start(0, 0)
        start(1, 1)

        def body(g, carry):
            for slot in range(2):
                j = 2 * g + slot
                pltpu.make_async_copy(
                    table_hbm.at[idx_v.at[j]], rows_v.at[slot], sems[slot]
                ).wait()
                pltpu.sync_copy(
                    rows_v.at[slot],
                    out_hbm.at[
                        pl.ds(j * B + wid * _CHUNK, _CHUNK), pl.ds(0, DIN)
                    ],
                )

                @pl.when(j + 2 < _NCHUNK)
                def _():
                    start(j + 2, slot)

            return carry

        lax.fori_loop(0, _NCHUNK // 2, body, 0)

    return _sc_gather


_BB = 256


def _tc_body(e_ref, sq_ref, w1_ref, b1_ref, w2_ref, b2_ref, o_ref, acc_ref):
    n = pl.program_id(1)

    @pl.when(n == 0)
    def _():
        acc_ref[...] = jnp.zeros((_BB, DOUT), jnp.float32)

    e = e_ref[:, :DIN]
    h = jnp.dot(e, w1_ref[...], preferred_element_type=jnp.float32)
    acc_ref[...] += jnp.maximum(h + b1_ref[...], 0.0)

    @pl.when(n == N - 1)
    def _():
        pooled = acc_ref[...] / sq_ref[...]
        o_ref[...] = (
            jnp.dot(pooled, w2_ref[...], preferred_element_type=jnp.float32)
            + b2_ref[...]
        )


def _tc_mlp(e, sq2, W1, b1, W2, b2):
    nb = B // _BB
    return pl.pallas_call(
        _tc_body,
        grid=(nb, N),
        in_specs=[
            pl.BlockSpec((_BB, _EW), lambda i, n: (n * nb + i, 0)),
            pl.BlockSpec((_BB, 1), lambda i, n: (i, 0)),
            pl.BlockSpec((DIN, DOUT), lambda i, n: (0, 0)),
            pl.BlockSpec((1, DOUT), lambda i, n: (0, 0)),
            pl.BlockSpec((DOUT, DOUT), lambda i, n: (0, 0)),
            pl.BlockSpec((1, DOUT), lambda i, n: (0, 0)),
        ],
        out_specs=pl.BlockSpec((_BB, DOUT), lambda i, n: (i, 0)),
        out_shape=jax.ShapeDtypeStruct((B, DOUT), jnp.float32),
        scratch_shapes=[pltpu.VMEM((_BB, DOUT), jnp.float32)],
    )(e, sq2, W1, b1, W2, b2)


def kernel(x, sq_lengths, weight, W1, b1, W2, b2):
    x3d = x.reshape(_NW, _CHUNK, N)
    e = _make_sc_gather()(x3d, weight)
    return _tc_mlp(
        e,
        sq_lengths.reshape(B, 1),
        W1,
        b1.reshape(1, DOUT),
        W2,
        b2.reshape(1, DOUT),
    )


# speedup vs baseline: 1.2423x; 1.2423x over previous
import functools

import jax
import jax.numpy as jnp
from jax import lax
from jax.experimental import pallas as pl
from jax.experimental.pallas import tpu as pltpu
from jax.experimental.pallas import tpu_sc as plsc

B, N, V, DIN, DOUT = 4096, 50, 1000000, 64, 64

_NC, _NS = 2, 16
_NW = _NC * _NS
_ROWS = B * N
_PER_W = _ROWS // _NW
_XC = 128
_NJ = _PER_W // _XC
_CH = 32
_QC = _XC // _CH
_NCHUNK = _PER_W // _CH
_SLABS = V // 8
_EROWS = _ROWS // 2
_EW = 2 * DIN


_RING = 8
_CHT = 128
_NCH = _PER_W // _CHT


@functools.cache
def _make_sc_gather():
    mesh = plsc.VectorSubcoreMesh(core_axis_name="c", subcore_axis_name="s")

    @functools.partial(
        pl.kernel,
        mesh=mesh,
        compiler_params=pltpu.CompilerParams(
            use_tc_tiling_on_sc=True, needs_layout_passes=False
        ),
        out_type=jax.ShapeDtypeStruct((_EROWS, _EW), jnp.float32),
        scratch_types=[
            pltpu.VMEM((_PER_W,), jnp.int32),
            pltpu.VMEM((_PER_W + 16,), jnp.int32),
            pltpu.VMEM((_PER_W + 16,), jnp.int32),
            pltpu.VMEM((_RING, 8, DIN), jnp.float32),
            pltpu.VMEM((2 * _CHT // 2, _EW), jnp.float32),
            [pltpu.SemaphoreType.DMA] * _RING,
            pltpu.SemaphoreType.DMA,
            pltpu.SemaphoreType.DMA,
        ],
    )
    def _sc_gather(
        x_hbm, w3_hbm, out_hbm,
        xv, slab_v, row_v, ring, ebuf,
        gsems, w0, w1,
    ):
        wid = lax.axis_index("s") * _NC + lax.axis_index("c")
        pltpu.sync_copy(x_hbm.at[wid], xv)

        def prep(k, carry):
            v = xv[pl.ds(16 * k, 16)]
            slab_v[pl.ds(16 * k, 16)] = lax.shift_right_logical(v, 3)
            row_v[pl.ds(16 * k, 16)] = lax.bitwise_and(v, 7)
            return carry

        lax.fori_loop(0, _PER_W // 16, prep, 0)

        wsems = (w0, w1)
        prow0 = wid * (_PER_W // 2)

        def sload(ref, t):
            return ref[pl.ds(t, 16)][0]

        def gdesc(t, r):
            return pltpu.make_async_copy(
                w3_hbm.at[sload(slab_v, t)], ring.at[r], gsems[r]
            )

        def wdesc(c, s):
            return pltpu.make_async_copy(
                ebuf.at[pl.ds((c % 2) * (_CHT // 2), _CHT // 2)],
                out_hbm.at[pl.ds(prow0 + c * (_CHT // 2), _CHT // 2)],
                wsems[s],
            )

        for r in range(_RING):
            gdesc(r, r).start()

        def body(g, carry):
            c = g // (_CHT // _RING)

            @pl.when((g % (_CHT // _RING) == 0) & (c >= 2))
            def _():
                @pl.when(c % 2 == 0)
                def _():
                    wdesc(c - 2, 0).wait()

                @pl.when(c % 2 == 1)
                def _():
                    wdesc(c - 2, 1).wait()

            for r in range(_RING):
                t = _RING * g + r
                gdesc(t, r).wait()
                rid = sload(row_v, t)
                m = lax.bitwise_and(
                    lax.shift_right_logical(t, 1), _CHT - 1
                )
                cb = (r % 2) * DIN
                for cc in range(DIN // 16):
                    ebuf[m, pl.ds(cb + 16 * cc, 16)] = ring[
                        r, rid, pl.ds(16 * cc, 16)
                    ]

                @pl.when(t + _RING < _PER_W)
                def _():
                    gdesc(t + _RING, r).start()

            @pl.when(g % (_CHT // _RING) == (_CHT // _RING) - 1)
            def _():
                @pl.when(c % 2 == 0)
                def _():
                    wdesc(c, 0).start()

                @pl.when(c % 2 == 1)
                def _():
                    wdesc(c, 1).start()

            return carry

        lax.fori_loop(0, _PER_W // _RING, body, 0)
        wdesc(_NCH - 2, 0).wait()
        wdesc(_NCH - 1, 1).wait()

    return _sc_gather


_BB = 128
_PR = _BB * (N // 2)


def _tc_body(e_ref, p_ref, sq_ref, w1_ref, b1_ref, w2_ref, b2_ref, o_ref):
    h2 = jnp.maximum(
        jnp.dot(e_ref[...], w1_ref[...], preferred_element_type=jnp.float32)
        + b1_ref[...],
        0.0,
    )
    pooled2 = jnp.dot(p_ref[...], h2, preferred_element_type=jnp.float32)
    y = jnp.dot(pooled2, w2_ref[...], preferred_element_type=jnp.float32)
    o_ref[...] = y / sq_ref[...] + b2_ref[...]


def _tc_mlp(e2, pmat, sq2, W1bd, b1x2, W2v, b2):
    nb = B // _BB
    return pl.pallas_call(
        _tc_body,
        grid=(nb,),
        in_specs=[
            pl.BlockSpec((_PR, _EW), lambda i: (i, 0)),
            pl.BlockSpec((_BB, _PR), lambda i: (0, 0)),
            pl.BlockSpec((_BB, 1), lambda i: (i, 0)),
            pl.BlockSpec((_EW, _EW), lambda i: (0, 0)),
            pl.BlockSpec((1, _EW), lambda i: (0, 0)),
            pl.BlockSpec((_EW, DOUT), lambda i: (0, 0)),
            pl.BlockSpec((1, DOUT), lambda i: (0, 0)),
        ],
        out_specs=pl.BlockSpec((_BB, DOUT), lambda i: (i, 0)),
        out_shape=jax.ShapeDtypeStruct((B, DOUT), jnp.float32),
    )(e2, pmat, sq2, W1bd, b1x2, W2v, b2)


def kernel(x, sq_lengths, weight, W1, b1, W2, b2):
    x2 = x.reshape(_NW, _PER_W)
    w3 = weight.reshape(_SLABS, 8, DIN)
    e2 = _make_sc_gather()(x2, w3)

    z = jnp.zeros((DIN, DOUT), jnp.float32)
    w1bd = jnp.concatenate(
        [
            jnp.concatenate([W1, z], axis=1),
            jnp.concatenate([z, W1], axis=1),
        ],
        axis=0,
    )
    b1x2 = jnp.concatenate([b1, b1]).reshape(1, _EW)
    w2v = jnp.concatenate([W2, W2], axis=0)
    pmat = (
        jnp.arange(_BB, dtype=jnp.int32)[:, None]
        == (jnp.arange(_PR, dtype=jnp.int32)[None, :] // (N // 2))
    ).astype(jnp.float32)
    return _tc_mlp(
        e2,
        pmat,
        sq_lengths.reshape(B, 1),
        w1bd,
        b1x2,
        w2v,
        b2.reshape(1, DOUT),
    )


# speedup vs baseline: 1.7315x; 1.3938x over previous
import functools

import jax
import jax.numpy as jnp
from jax import lax
from jax.experimental import pallas as pl
from jax.experimental.pallas import tpu as pltpu
from jax.experimental.pallas import tpu_sc as plsc

B, N, V, DIN, DOUT = 4096, 50, 1000000, 64, 64
_EW = 128

_TCB = 2048


def _tr_body(wt_ref, wp_ref):
    wp_ref[:, :DIN] = wt_ref[...].T


def _tc_transpose(wt):
    grid = ((V + _TCB - 1) // _TCB,)
    return pl.pallas_call(
        _tr_body,
        grid=grid,
        in_specs=[pl.BlockSpec((DIN, _TCB), lambda i: (0, i))],
        out_specs=pl.BlockSpec((_TCB, _EW), lambda i: (i, 0)),
        out_shape=jax.ShapeDtypeStruct((V, _EW), jnp.float32),
    )(wt)


_NC, _NS = 2, 16
_NW = _NC * _NS
_ROWS = B * N
_PER_W = _ROWS // _NW
_CHUNK = 128
_NCHUNK = _PER_W // _CHUNK


@functools.cache
def _make_sc_gather():
    mesh = plsc.VectorSubcoreMesh(core_axis_name="c", subcore_axis_name="s")

    @functools.partial(
        pl.kernel,
        mesh=mesh,
        compiler_params=pltpu.CompilerParams(
            use_tc_tiling_on_sc=True, needs_layout_passes=False
        ),
        out_type=jax.ShapeDtypeStruct((_ROWS, _EW), jnp.float32),
        scratch_types=[
            pltpu.VMEM((_PER_W,), jnp.int32),
            pltpu.VMEM((2, _CHUNK, _EW), jnp.float32),
            pltpu.SemaphoreType.DMA,
            pltpu.SemaphoreType.DMA,
        ],
    )
    def _sc_gather(x_hbm, wp_hbm, out_hbm, xv, rows_v, sem0, sem1):
        wid = lax.axis_index("s") * _NC + lax.axis_index("c")
        pltpu.sync_copy(x_hbm.at[wid], xv)
        base = wid * _PER_W
        sems = (sem0, sem1)

        def gdesc(j, slot):
            return pltpu.make_async_copy(
                wp_hbm.at[xv.at[pl.ds(j * _CHUNK, _CHUNK)]],
                rows_v.at[slot],
                sems[slot],
            )

        gdesc(0, 0).start()
        gdesc(1, 1).start()

        def body(g, carry):
            for slot in range(2):
                j = 2 * g + slot
                gdesc(j, slot).wait()
                pltpu.sync_copy(
                    rows_v.at[slot],
                    out_hbm.at[pl.ds(base + j * _CHUNK, _CHUNK)],
                )

                @pl.when(j + 2 < _NCHUNK)
                def _():
                    gdesc(j + 2, slot).start()

            return carry

        lax.fori_loop(0, _NCHUNK // 2, body, 0)

    return _sc_gather


_BB = 64
_TR = _BB * N


def _tc_body(e_ref, p_ref, sq_ref, w1_ref, b1_ref, w2_ref, b2_ref, o_ref):
    e = e_ref[:, :DIN]
    h = jnp.maximum(
        jnp.dot(e, w1_ref[...], preferred_element_type=jnp.float32)
        + b1_ref[...],
        0.0,
    )
    pooled = jnp.dot(p_ref[...], h, preferred_element_type=jnp.float32)
    y = jnp.dot(pooled, w2_ref[...], preferred_element_type=jnp.float32)
    o_ref[...] = y / sq_ref[...] + b2_ref[...]


def _tc_mlp(e2, pmat, sq2, W1, b1, W2, b2):
    nb = B // _BB
    return pl.pallas_call(
        _tc_body,
        grid=(nb,),
        in_specs=[
            pl.BlockSpec((_TR, _EW), lambda i: (i, 0)),
            pl.BlockSpec((_BB, _TR), lambda i: (0, 0)),
            pl.BlockSpec((_BB, 1), lambda i: (i, 0)),
            pl.BlockSpec((DIN, DOUT), lambda i: (0, 0)),
            pl.BlockSpec((1, DOUT), lambda i: (0, 0)),
            pl.BlockSpec((DOUT, DOUT), lambda i: (0, 0)),
            pl.BlockSpec((1, DOUT), lambda i: (0, 0)),
        ],
        out_specs=pl.BlockSpec((_BB, DOUT), lambda i: (i, 0)),
        out_shape=jax.ShapeDtypeStruct((B, DOUT), jnp.float32),
    )(e2, pmat, sq2, W1, b1, W2, b2)


def kernel(x, sq_lengths, weight, W1, b1, W2, b2):
    wp = _tc_transpose(weight.T)
    x2 = x.reshape(_NW, _PER_W)
    e2 = _make_sc_gather()(x2, wp)
    pmat = (
        jnp.arange(_BB, dtype=jnp.int32)[:, None]
        == (jnp.arange(_TR, dtype=jnp.int32)[None, :] // N)
    ).astype(jnp.float32)
    return _tc_mlp(
        e2,
        pmat,
        sq_lengths.reshape(B, 1),
        W1,
        b1.reshape(1, DOUT),
        W2,
        b2.reshape(1, DOUT),
    )


# speedup vs baseline: 2.5176x; 1.4540x over previous
import functools

import jax
import jax.numpy as jnp
from jax import lax
from jax.experimental import pallas as pl
from jax.experimental.pallas import tpu as pltpu
from jax.experimental.pallas import tpu_sc as plsc

B, N, V, DIN, DOUT = 4096, 50, 1000000, 64, 64
_EW = 128

_TCB = 8192


def _tr_body(wt_ref, wp_ref):
    wp_ref[:, :DIN] = wt_ref[...].T


def _tc_transpose(wt):
    grid = ((V + _TCB - 1) // _TCB,)
    return pl.pallas_call(
        _tr_body,
        grid=grid,
        in_specs=[pl.BlockSpec((DIN, _TCB), lambda i: (0, i))],
        out_specs=pl.BlockSpec((_TCB, _EW), lambda i: (i, 0)),
        out_shape=jax.ShapeDtypeStruct((V, _EW), jnp.float32),
    )(wt)


_NC, _NS = 2, 16
_NW = _NC * _NS
_ROWS = B * N
_PER_W = _ROWS // _NW
_CHUNK = 128
_NCHUNK = _PER_W // _CHUNK


@functools.cache
def _make_sc_gather():
    mesh = plsc.VectorSubcoreMesh(core_axis_name="c", subcore_axis_name="s")

    @functools.partial(
        pl.kernel,
        mesh=mesh,
        compiler_params=pltpu.CompilerParams(
            use_tc_tiling_on_sc=True, needs_layout_passes=False
        ),
        out_type=jax.ShapeDtypeStruct((_ROWS, _EW), jnp.float32),
        scratch_types=[
            pltpu.VMEM((_PER_W,), jnp.int32),
            pltpu.VMEM((2, _CHUNK, _EW), jnp.float32),
            pltpu.SemaphoreType.DMA,
            pltpu.SemaphoreType.DMA,
        ],
    )
    def _sc_gather(x_hbm, wp_hbm, out_hbm, xv, rows_v, sem0, sem1):
        wid = lax.axis_index("s") * _NC + lax.axis_index("c")
        pltpu.sync_copy(x_hbm.at[wid], xv)
        base = wid * _PER_W
        sems = (sem0, sem1)

        def gdesc(j, slot):
            return pltpu.make_async_copy(
                wp_hbm.at[xv.at[pl.ds(j * _CHUNK, _CHUNK)]],
                rows_v.at[slot],
                sems[slot],
            )

        gdesc(0, 0).start()
        gdesc(1, 1).start()

        def body(g, carry):
            for slot in range(2):
                j = 2 * g + slot
                gdesc(j, slot).wait()
                pltpu.sync_copy(
                    rows_v.at[slot],
                    out_hbm.at[pl.ds(base + j * _CHUNK, _CHUNK)],
                )

                @pl.when(j + 2 < _NCHUNK)
                def _():
                    gdesc(j + 2, slot).start()

            return carry

        lax.fori_loop(0, _NCHUNK // 2, body, 0)

    return _sc_gather


_BB = 64
_TR = _BB * N


def _tc_body(e_ref, p_ref, sq_ref, w1_ref, b1_ref, w2_ref, b2_ref, o_ref):
    e = e_ref[:, :DIN]
    h = jnp.maximum(
        jnp.dot(e, w1_ref[...], preferred_element_type=jnp.float32)
        + b1_ref[...],
        0.0,
    )
    pooled = jnp.dot(p_ref[...], h, preferred_element_type=jnp.float32)
    y = jnp.dot(pooled, w2_ref[...], preferred_element_type=jnp.float32)
    o_ref[...] = y / sq_ref[...] + b2_ref[...]


def _tc_mlp(e2, pmat, sq2, W1, b1, W2, b2):
    nb = B // _BB
    return pl.pallas_call(
        _tc_body,
        grid=(nb,),
        in_specs=[
            pl.BlockSpec((_TR, _EW), lambda i: (i, 0)),
            pl.BlockSpec((_BB, _TR), lambda i: (0, 0)),
            pl.BlockSpec((_BB, 1), lambda i: (i, 0)),
            pl.BlockSpec((DIN, DOUT), lambda i: (0, 0)),
            pl.BlockSpec((1, DOUT), lambda i: (0, 0)),
            pl.BlockSpec((DOUT, DOUT), lambda i: (0, 0)),
            pl.BlockSpec((1, DOUT), lambda i: (0, 0)),
        ],
        out_specs=pl.BlockSpec((_BB, DOUT), lambda i: (i, 0)),
        out_shape=jax.ShapeDtypeStruct((B, DOUT), jnp.float32),
    )(e2, pmat, sq2, W1, b1, W2, b2)


def kernel(x, sq_lengths, weight, W1, b1, W2, b2):
    wp = _tc_transpose(weight.T)
    x2 = x.reshape(_NW, _PER_W)
    e2 = _make_sc_gather()(x2, wp)
    pmat = (
        jnp.arange(_BB, dtype=jnp.int32)[:, None]
        == (jnp.arange(_TR, dtype=jnp.int32)[None, :] // N)
    ).astype(jnp.float32)
    return _tc_mlp(
        e2,
        pmat,
        sq_lengths.reshape(B, 1),
        W1,
        b1.reshape(1, DOUT),
        W2,
        b2.reshape(1, DOUT),
    )
